# Initial kernel scaffold; baseline (speedup 1.0000x reference)
import functools

import jax
import jax.numpy as jnp
from jax import lax
from jax.experimental import pallas as pl
from jax.experimental.pallas import tpu as pltpu
from jax.experimental.pallas import tpu_sc as plsc

N = 10000
E = 320000
D = 128

NC = 2
NS = 16
NW = NC * NS

EPT = E // NW
C = 80
NCHUNK = EPT // C
RPT = N // NS
ZR = 125

BR = 1250


def _tc_pre_body(x_ref, w_ref, att2_ref, h_ref, asd_ref):
    h = jnp.dot(x_ref[...], w_ref[...], preferred_element_type=jnp.float32)
    h_ref[...] = h
    asd_ref[...] = jnp.dot(h, att2_ref[...], preferred_element_type=jnp.float32)


def _tc_pre(x, W, att2):
    return pl.pallas_call(
        _tc_pre_body,
        grid=(N // BR,),
        in_specs=[
            pl.BlockSpec((BR, D), lambda i: (i, 0)),
            pl.BlockSpec((D, D), lambda i: (0, 0)),
            pl.BlockSpec((D, 2), lambda i: (0, 0)),
        ],
        out_specs=[
            pl.BlockSpec((BR, D), lambda i: (i, 0)),
            pl.BlockSpec((BR, 2), lambda i: (i, 0)),
        ],
        out_shape=[
            jax.ShapeDtypeStruct((N, D), jnp.float32),
            jax.ShapeDtypeStruct((N, 2), jnp.float32),
        ],
    )(x, W, att2)


def _sc_agg_body(h_hbm, asd_hbm, src_hbm, dst_hbm, S_out, den_out,
                 asd_v, sidx_v, didx_v, ee_v, rows_v, zbuf_v, zden_v,
                 S_sh, den_sh, sem):
    c = lax.axis_index("c")
    s = lax.axis_index("s")
    wid = c * NS + s

    z16f = jnp.zeros((16,), jnp.float32)

    def zrow(r, carry):
        for k in range(D // 16):
            zbuf_v[r, pl.ds(k * 16, 16)] = z16f
        return carry
    lax.fori_loop(0, ZR, zrow, 0)

    def zden(i, carry):
        zden_v[pl.ds(i * 16, 16)] = z16f
        return carry
    lax.fori_loop(0, 640 // 16, zden, 0)

    for j in range(RPT // ZR):
        pltpu.sync_copy(zbuf_v, S_sh.at[pl.ds(s * RPT + j * ZR, ZR)])
    dstart = jnp.minimum(s * 640, N - 640)
    pltpu.sync_copy(zden_v, den_sh.at[pl.ds(dstart, 640)])

    pltpu.sync_copy(asd_hbm, asd_v)
    plsc.subcore_barrier()

    zero16 = jnp.zeros((16,), jnp.int32)
    one16 = jnp.ones((16,), jnp.int32)

    def chunk(g, carry):
        base = wid * EPT + g * C
        pltpu.sync_copy(src_hbm.at[pl.ds(base, C)], sidx_v)
        pltpu.sync_copy(dst_hbm.at[pl.ds(base, C)], didx_v)
        cp = pltpu.async_copy(h_hbm.at[sidx_v], rows_v, sem)
        for k in range(C // 16):
            s16 = sidx_v[pl.ds(k * 16, 16)]
            d16 = didx_v[pl.ds(k * 16, 16)]
            a_s = plsc.load_gather(asd_v, [s16, zero16])
            a_d = plsc.load_gather(asd_v, [d16, one16])
            e = a_s + a_d
            e = jnp.where(e >= 0.0, e, 0.2 * e)
            ee_v[pl.ds(k * 16, 16)] = jnp.exp(e)
        cp.wait()

        def srow(r, carry2):
            eb = plsc.load_gather(ee_v, [jnp.full((16,), r, jnp.int32)])
            for k in range(D // 16):
                rows_v[r, pl.ds(k * 16, 16)] = rows_v[r, pl.ds(k * 16, 16)] * eb
            return carry2
        lax.fori_loop(0, C, srow, 0)

        pltpu.sync_copy(ee_v, den_sh.at[didx_v], add=True)
        pltpu.sync_copy(rows_v, S_sh.at[didx_v], add=True)
        return carry
    lax.fori_loop(0, NCHUNK, chunk, 0)

    plsc.subcore_barrier()

    pltpu.sync_copy(S_sh.at[pl.ds(s * RPT, RPT)], S_out.at[c, pl.ds(s * RPT, RPT)])
    @pl.when(s < NS - 1)
    def _():
        pltpu.sync_copy(den_sh.at[pl.ds(s * 624, 624)],
                        den_out.at[c, pl.ds(s * 624, 624)])
    @pl.when(s == NS - 1)
    def _():
        pltpu.sync_copy(den_sh.at[pl.ds(N - 640, 640)],
                        den_out.at[c, pl.ds(N - 640, 640)])


_sc_agg = functools.partial(
    pl.kernel,
    out_type=[
        jax.ShapeDtypeStruct((NC, N, D), jnp.float32),
        jax.ShapeDtypeStruct((NC, N), jnp.float32),
    ],
    mesh=plsc.VectorSubcoreMesh(
        core_axis_name="c", subcore_axis_name="s",
        num_cores=NC, num_subcores=NS),
    scratch_types=[
        pltpu.VMEM((N, 2), jnp.float32),
        pltpu.VMEM((C,), jnp.int32),
        pltpu.VMEM((C,), jnp.int32),
        pltpu.VMEM((C,), jnp.float32),
        pltpu.VMEM((C, D), jnp.float32),
        pltpu.VMEM((ZR, D), jnp.float32),
        pltpu.VMEM((640,), jnp.float32),
        pltpu.VMEM_SHARED((N, D), jnp.float32),
        pltpu.VMEM_SHARED((N,), jnp.float32),
        pltpu.SemaphoreType.DMA,
    ],
)(_sc_agg_body)


def _tc_post_body(S_ref, den_ref, bias_ref, out_ref):
    Ssum = S_ref[0] + S_ref[1]
    den = den_ref[0] + den_ref[1] + 1e-16
    out_ref[...] = jnp.tanh(Ssum / den[:, None] + bias_ref[...])


def _tc_post(S, den_p, bias2):
    return pl.pallas_call(
        _tc_post_body,
        grid=(N // BR,),
        in_specs=[
            pl.BlockSpec((NC, BR, D), lambda i: (0, i, 0)),
            pl.BlockSpec((NC, BR), lambda i: (0, i)),
            pl.BlockSpec((1, D), lambda i: (0, 0)),
        ],
        out_specs=pl.BlockSpec((BR, D), lambda i: (i, 0)),
        out_shape=jax.ShapeDtypeStruct((N, D), jnp.float32),
    )(S, den_p, bias2)


def kernel(x, edge_index, W, att_src, att_dst, bias):
    src = edge_index[0].astype(jnp.int32)
    dst = edge_index[1].astype(jnp.int32)
    att2 = jnp.stack([att_src, att_dst], axis=1)
    h, asd = _tc_pre(x, W, att2)
    S, den_p = _sc_agg(h, asd, src, dst)
    return _tc_post(S, den_p, bias.reshape(1, D))


# speedup vs baseline: 14.2481x; 14.2481x over previous
import functools

import jax
import jax.numpy as jnp
from jax import lax
from jax.experimental import pallas as pl
from jax.experimental.pallas import tpu as pltpu
from jax.experimental.pallas import tpu_sc as plsc

N = 10000
E = 320000
D = 128
D2 = D // 2

NC = 2
NS = 16

EPT = E // NS
C = 80
NCHUNK = EPT // C
ZR = 64


def _tc_pre_body(x_ref, w_ref, att2_ref, h_ref, asd_ref):
    h = jnp.dot(x_ref[...], w_ref[...], preferred_element_type=jnp.float32)
    h_ref[...] = h
    asd_ref[...] = jnp.dot(h, att2_ref[...], preferred_element_type=jnp.float32)


def _tc_pre(x, W, att2):
    return pl.pallas_call(
        _tc_pre_body,
        out_shape=[
            jax.ShapeDtypeStruct((N, D), jnp.float32),
            jax.ShapeDtypeStruct((N, 2), jnp.float32),
        ],
    )(x, W, att2)


def _sc_agg_body(ht_hbm, asd_hbm, src_hbm, dst_hbm, S_out, den_out,
                 asd_v, sidx_v, sidx2_v, didx_v, ee_v, rows_v, zbuf_v,
                 zden_v, S_sh, den_sh, sem):
    c = lax.axis_index("c")
    s = lax.axis_index("s")

    z16f = jnp.zeros((16,), jnp.float32)

    def zrow(r, carry):
        for k in range(D2 // 16):
            zbuf_v[r, pl.ds(k * 16, 16)] = z16f
        return carry
    lax.fori_loop(0, ZR, zrow, 0)

    def zden(i, carry):
        zden_v[pl.ds(i * 16, 16)] = z16f
        return carry
    lax.fori_loop(0, 640 // 16, zden, 0)

    dstart = jnp.minimum(s * 640, N - 640)
    for j in range(640 // ZR):
        pltpu.sync_copy(zbuf_v, S_sh.at[pl.ds(dstart + j * ZR, ZR)])
    pltpu.sync_copy(zden_v, den_sh.at[pl.ds(dstart, 640)])

    pltpu.sync_copy(asd_hbm, asd_v)
    plsc.subcore_barrier()

    zero16 = jnp.zeros((16,), jnp.int32)
    one16 = jnp.ones((16,), jnp.int32)

    def chunk(g, carry):
        base = s * EPT + g * C
        pltpu.sync_copy(src_hbm.at[pl.ds(base, C)], sidx_v)
        pltpu.sync_copy(dst_hbm.at[pl.ds(base, C)], didx_v)
        s16s = []
        for k in range(C // 16):
            s16 = sidx_v[pl.ds(k * 16, 16)]
            sidx2_v[pl.ds(k * 16, 16)] = s16 * 2 + c
            s16s.append(s16)
        cp = pltpu.async_copy(ht_hbm.at[sidx2_v], rows_v, sem)
        for k in range(C // 16):
            d16 = didx_v[pl.ds(k * 16, 16)]
            a_s = plsc.load_gather(asd_v, [s16s[k], zero16])
            a_d = plsc.load_gather(asd_v, [d16, one16])
            e = a_s + a_d
            e = jnp.where(e >= 0.0, e, 0.2 * e)
            ee_v[pl.ds(k * 16, 16)] = jnp.exp(e)
        cp.wait()

        def srow(r, carry2):
            eb = plsc.load_gather(ee_v, [jnp.full((16,), r, jnp.int32)])
            for k in range(D2 // 16):
                rows_v[r, pl.ds(k * 16, 16)] = rows_v[r, pl.ds(k * 16, 16)] * eb
            return carry2
        lax.fori_loop(0, C, srow, 0)

        @pl.when(c == 0)
        def _():
            pltpu.sync_copy(ee_v, den_sh.at[didx_v], add=True)
        pltpu.sync_copy(rows_v, S_sh.at[didx_v], add=True)
        return carry
    lax.fori_loop(0, NCHUNK, chunk, 0)

    plsc.subcore_barrier()

    start = jnp.minimum(s * 624, N - 640)
    nrows = jnp.where(s < NS - 1, 624, 640)
    del nrows

    @pl.when(s < NS - 1)
    def _():
        pltpu.sync_copy(S_sh.at[pl.ds(s * 624, 624)],
                        S_out.at[pl.ds(s * 624, 624), c])

    @pl.when(s == NS - 1)
    def _():
        pltpu.sync_copy(S_sh.at[pl.ds(N - 640, 640)],
                        S_out.at[pl.ds(N - 640, 640), c])

    @pl.when(c == 0)
    def _():
        @pl.when(s < NS - 1)
        def _():
            pltpu.sync_copy(den_sh.at[pl.ds(s * 624, 624)],
                            den_out.at[0, pl.ds(s * 624, 624)])
        @pl.when(s == NS - 1)
        def _():
            pltpu.sync_copy(den_sh.at[pl.ds(N - 640, 640)],
                            den_out.at[0, pl.ds(N - 640, 640)])


_sc_agg = functools.partial(
    pl.kernel,
    out_type=[
        jax.ShapeDtypeStruct((N, NC, D2), jnp.float32),
        jax.ShapeDtypeStruct((1, N), jnp.float32),
    ],
    mesh=plsc.VectorSubcoreMesh(
        core_axis_name="c", subcore_axis_name="s",
        num_cores=NC, num_subcores=NS),
    scratch_types=[
        pltpu.VMEM((N, 2), jnp.float32),
        pltpu.VMEM((C,), jnp.int32),
        pltpu.VMEM((C,), jnp.int32),
        pltpu.VMEM((C,), jnp.int32),
        pltpu.VMEM((C,), jnp.float32),
        pltpu.VMEM((C, D2), jnp.float32),
        pltpu.VMEM((ZR, D2), jnp.float32),
        pltpu.VMEM((640,), jnp.float32),
        pltpu.VMEM_SHARED((N, D2), jnp.float32),
        pltpu.VMEM_SHARED((N,), jnp.float32),
        pltpu.SemaphoreType.DMA,
    ],
    compiler_params=pltpu.CompilerParams(
        use_tc_tiling_on_sc=False, needs_layout_passes=False),
)(_sc_agg_body)


def _tc_post_body(S_ref, den_ref, bias_ref, out_ref):
    den = den_ref[...] + 1e-16
    out_ref[...] = jnp.tanh(S_ref[...] / den + bias_ref[...])


def _tc_post(S, den_col, bias2):
    return pl.pallas_call(
        _tc_post_body,
        out_shape=jax.ShapeDtypeStruct((N, D), jnp.float32),
    )(S, den_col, bias2)


def kernel(x, edge_index, W, att_src, att_dst, bias):
    src = edge_index[0].astype(jnp.int32)
    dst = edge_index[1].astype(jnp.int32)
    att2 = jnp.stack([att_src, att_dst], axis=1)
    h, asd = _tc_pre(x, W, att2)
    ht = h.reshape(2 * N, D2)
    S, den_p = _sc_agg(ht, asd, src, dst)
    return _tc_post(S.reshape(N, D), den_p.reshape(N, 1), bias.reshape(1, D))


# speedup vs baseline: 17.2073x; 1.2077x over previous
import functools

import jax
import jax.numpy as jnp
from jax import lax
from jax.experimental import pallas as pl
from jax.experimental.pallas import tpu as pltpu
from jax.experimental.pallas import tpu_sc as plsc

N = 10000
E = 320000
D = 128
D2 = D // 2

NC = 2
NS = 16

EPT = E // NS
C = 80
NCHUNK = EPT // C


def _tc_pre_body(x_ref, w_ref, att2_ref, h_ref, asd_ref):
    h = jnp.dot(x_ref[...], w_ref[...], preferred_element_type=jnp.float32)
    h_ref[...] = h
    asd_ref[...] = jnp.dot(h, att2_ref[...], preferred_element_type=jnp.float32)


def _tc_pre(x, W, att2):
    return pl.pallas_call(
        _tc_pre_body,
        out_shape=[
            jax.ShapeDtypeStruct((N, D), jnp.float32),
            jax.ShapeDtypeStruct((N, 2), jnp.float32),
        ],
    )(x, W, att2)


def _sc_agg_body(ht_hbm, asd_hbm, src_hbm, dst_hbm, S0_out, S1_out, den_out,
                 asd_v, sidx_v, didx_v, ee_v,
                 rowsA_v, S_sh, den_sh, sem):
    c = lax.axis_index("c")
    s = lax.axis_index("s")

    z16f = jnp.zeros((16,), jnp.float32)

    def zrow(r, carry):
        for k in range(D2 // 16):
            rowsA_v[r, pl.ds(k * 16, 16)] = z16f
        return carry
    lax.fori_loop(0, C, zrow, 0)
    for k in range(C // 16):
        ee_v[pl.ds(k * 16, 16)] = z16f

    dstart = jnp.minimum(s * 640, N - 640)
    for j in range(640 // C):
        pltpu.sync_copy(rowsA_v, S_sh.at[pl.ds(dstart + j * C, C)])
        pltpu.sync_copy(ee_v, den_sh.at[pl.ds(dstart + j * C, C)])

    pltpu.sync_copy(asd_hbm, asd_v)
    plsc.subcore_barrier()

    zero16 = jnp.zeros((16,), jnp.int32)
    one16 = jnp.ones((16,), jnp.int32)

    def issue(g, sidx_v, rows_v):
        base = s * EPT + g * C
        pltpu.sync_copy(src_hbm.at[pl.ds(base, C)], sidx_v)
        s16s = []
        for k in range(C // 16):
            s16 = sidx_v[pl.ds(k * 16, 16)]
            s16s.append(s16)
            sidx_v[pl.ds(k * 16, 16)] = s16 * 2 + c
        return pltpu.async_copy(ht_hbm.at[sidx_v], rows_v, sem), s16s

    def prep(g, s16s):
        base = s * EPT + g * C
        pltpu.sync_copy(dst_hbm.at[pl.ds(base, C)], didx_v)
        for k in range(C // 16):
            d16 = didx_v[pl.ds(k * 16, 16)]
            a_s = plsc.load_gather(asd_v, [s16s[k], zero16])
            a_d = plsc.load_gather(asd_v, [d16, one16])
            e = a_s + a_d
            e = jnp.where(e >= 0.0, e, 0.2 * e)
            ee_v[pl.ds(k * 16, 16)] = jnp.exp(e)

    def drain(rows_v):
        def srow(r, carry2):
            eb = plsc.load_gather(ee_v, [jnp.full((16,), r, jnp.int32)])
            for k in range(D2 // 16):
                rows_v[r, pl.ds(k * 16, 16)] = rows_v[r, pl.ds(k * 16, 16)] * eb
            return carry2
        lax.fori_loop(0, C, srow, 0)

        @pl.when(c == 0)
        def _():
            pltpu.sync_copy(ee_v, den_sh.at[didx_v], add=True)
        pltpu.sync_copy(rows_v, S_sh.at[didx_v], add=True)

    def chunk(g, carry):
        cp, s16 = issue(g, sidx_v, rowsA_v)
        prep(g, s16)
        cp.wait()
        drain(rowsA_v)
        return carry
    lax.fori_loop(0, NCHUNK, chunk, 0)

    plsc.subcore_barrier()

    def strip_writes(lo, n):
        @pl.when(c == 0)
        def _():
            pltpu.sync_copy(S_sh.at[pl.ds(lo, n)], S0_out.at[pl.ds(lo, n)])
            pltpu.sync_copy(den_sh.at[pl.ds(lo, n)], den_out.at[pl.ds(lo, n)])

        @pl.when(c == 1)
        def _():
            pltpu.sync_copy(S_sh.at[pl.ds(lo, n)], S1_out.at[pl.ds(lo, n)])

    @pl.when(s < NS - 1)
    def _():
        strip_writes(s * 624, 624)

    @pl.when(s == NS - 1)
    def _():
        strip_writes(N - 640, 640)


_sc_agg = functools.partial(
    pl.kernel,
    out_type=[
        jax.ShapeDtypeStruct((N, D2), jnp.float32),
        jax.ShapeDtypeStruct((N, D2), jnp.float32),
        jax.ShapeDtypeStruct((N,), jnp.float32),
    ],
    mesh=plsc.VectorSubcoreMesh(
        core_axis_name="c", subcore_axis_name="s",
        num_cores=NC, num_subcores=NS),
    scratch_types=[
        pltpu.VMEM((N, 2), jnp.float32),
        pltpu.VMEM((C,), jnp.int32),
        pltpu.VMEM((C,), jnp.int32),
        pltpu.VMEM((C,), jnp.float32),
        pltpu.VMEM((C, D2), jnp.float32),
        pltpu.VMEM_SHARED((N, D2), jnp.float32),
        pltpu.VMEM_SHARED((N,), jnp.float32),
        pltpu.SemaphoreType.DMA,
    ],
    compiler_params=pltpu.CompilerParams(
        use_tc_tiling_on_sc=False, needs_layout_passes=False),
)(_sc_agg_body)


def _tc_post_body(S0_ref, S1_ref, den_ref, bias_ref, out_ref):
    den = den_ref[...] + 1e-16
    out_ref[:, :D2] = jnp.tanh(S0_ref[...] / den + bias_ref[:, :D2])
    out_ref[:, D2:] = jnp.tanh(S1_ref[...] / den + bias_ref[:, D2:])


def _tc_post(S0, S1, den_col, bias2):
    return pl.pallas_call(
        _tc_post_body,
        out_shape=jax.ShapeDtypeStruct((N, D), jnp.float32),
    )(S0, S1, den_col, bias2)


def kernel(x, edge_index, W, att_src, att_dst, bias):
    src = edge_index[0].astype(jnp.int32)
    dst = edge_index[1].astype(jnp.int32)
    att2 = jnp.stack([att_src, att_dst], axis=1)
    h, asd = _tc_pre(x, W, att2)
    ht = h.reshape(2 * N, D2)
    S0, S1, den_p = _sc_agg(ht, asd, src, dst)
    return _tc_post(S0, S1, den_p.reshape(N, 1), bias.reshape(1, D))


# speedup vs baseline: 19.9689x; 1.1605x over previous
import functools

import jax
import jax.numpy as jnp
from jax import lax
from jax.experimental import pallas as pl
from jax.experimental.pallas import tpu as pltpu
from jax.experimental.pallas import tpu_sc as plsc

N = 10000
E = 320000
D = 128
D2 = D // 2

NC = 2
NS = 16

EPT = E // NS
C = 80
BLK = 2000


def _tc_pre_body(x_ref, w_ref, att2_ref, h_ref, asd_ref):
    h = jnp.dot(x_ref[...], w_ref[...], preferred_element_type=jnp.float32)
    h_ref[...] = h
    asd_ref[...] = jnp.dot(h, att2_ref[...], preferred_element_type=jnp.float32)


def _tc_pre(x, W, att2):
    return pl.pallas_call(
        _tc_pre_body,
        out_shape=[
            jax.ShapeDtypeStruct((N, D), jnp.float32),
            jax.ShapeDtypeStruct((N, 2), jnp.float32),
        ],
    )(x, W, att2)


def _sc_agg_body(ht_hbm, asd_hbm, src_hbm, dst_hbm, S0_out, S1_out, den_out,
                 asd_v, sblk_v, dblk_v, sidx_v, didx_v, ee_v,
                 rowsA_v, S_sh, den_sh, sem):
    c = lax.axis_index("c")
    s = lax.axis_index("s")

    z16f = jnp.zeros((16,), jnp.float32)

    def zrow(r, carry):
        for k in range(D2 // 16):
            rowsA_v[r, pl.ds(k * 16, 16)] = z16f
        return carry
    lax.fori_loop(0, C, zrow, 0)
    for k in range(C // 16):
        ee_v[pl.ds(k * 16, 16)] = z16f

    dstart = jnp.minimum(s * 640, N - 640)
    for j in range(640 // C):
        pltpu.sync_copy(rowsA_v, S_sh.at[pl.ds(dstart + j * C, C)])
        pltpu.sync_copy(ee_v, den_sh.at[pl.ds(dstart + j * C, C)])

    pltpu.sync_copy(asd_hbm, asd_v)
    plsc.subcore_barrier()

    zero16 = jnp.zeros((16,), jnp.int32)
    one16 = jnp.ones((16,), jnp.int32)

    def chunk(j, carry):
        off = j * C
        s16s = []
        d16s = []
        for k in range(C // 16):
            s16 = sblk_v[pl.ds(off + k * 16, 16)]
            s16s.append(s16)
            sidx_v[pl.ds(k * 16, 16)] = s16 * 2 + c
        cp = pltpu.async_copy(ht_hbm.at[sidx_v], rowsA_v, sem)
        for k in range(C // 16):
            d16 = dblk_v[pl.ds(off + k * 16, 16)]
            d16s.append(d16)
            didx_v[pl.ds(k * 16, 16)] = d16
        for k in range(C // 16):
            a_s = plsc.load_gather(asd_v, [s16s[k], zero16])
            a_d = plsc.load_gather(asd_v, [d16s[k], one16])
            e = a_s + a_d
            e = jnp.where(e >= 0.0, e, 0.2 * e)
            ee_v[pl.ds(k * 16, 16)] = jnp.exp(e)
        cp.wait()

        def srow(r, carry2):
            eb = plsc.load_gather(ee_v, [jnp.full((16,), r, jnp.int32)])
            for k in range(D2 // 16):
                rowsA_v[r, pl.ds(k * 16, 16)] = rowsA_v[r, pl.ds(k * 16, 16)] * eb
            return carry2
        lax.fori_loop(0, C, srow, 0)

        @pl.when(c == 0)
        def _():
            pltpu.sync_copy(ee_v, den_sh.at[didx_v], add=True)
        pltpu.sync_copy(rowsA_v, S_sh.at[didx_v], add=True)
        return carry

    def block(b, carry):
        base = s * EPT + b * BLK
        pltpu.sync_copy(src_hbm.at[pl.ds(base, BLK)], sblk_v)
        pltpu.sync_copy(dst_hbm.at[pl.ds(base, BLK)], dblk_v)
        lax.fori_loop(0, BLK // C, chunk, carry)
        return carry
    lax.fori_loop(0, EPT // BLK, block, 0)

    plsc.subcore_barrier()

    def strip_writes(lo, n):
        @pl.when(c == 0)
        def _():
            pltpu.sync_copy(S_sh.at[pl.ds(lo, n)], S0_out.at[pl.ds(lo, n)])
            pltpu.sync_copy(den_sh.at[pl.ds(lo, n)], den_out.at[pl.ds(lo, n)])

        @pl.when(c == 1)
        def _():
            pltpu.sync_copy(S_sh.at[pl.ds(lo, n)], S1_out.at[pl.ds(lo, n)])

    @pl.when(s < NS - 1)
    def _():
        strip_writes(s * 624, 624)

    @pl.when(s == NS - 1)
    def _():
        strip_writes(N - 640, 640)


_sc_agg = functools.partial(
    pl.kernel,
    out_type=[
        jax.ShapeDtypeStruct((N, D2), jnp.float32),
        jax.ShapeDtypeStruct((N, D2), jnp.float32),
        jax.ShapeDtypeStruct((N,), jnp.float32),
    ],
    mesh=plsc.VectorSubcoreMesh(
        core_axis_name="c", subcore_axis_name="s",
        num_cores=NC, num_subcores=NS),
    scratch_types=[
        pltpu.VMEM((N, 2), jnp.float32),
        pltpu.VMEM((BLK,), jnp.int32),
        pltpu.VMEM((BLK,), jnp.int32),
        pltpu.VMEM((C,), jnp.int32),
        pltpu.VMEM((C,), jnp.int32),
        pltpu.VMEM((C,), jnp.float32),
        pltpu.VMEM((C, D2), jnp.float32),
        pltpu.VMEM_SHARED((N, D2), jnp.float32),
        pltpu.VMEM_SHARED((N,), jnp.float32),
        pltpu.SemaphoreType.DMA,
    ],
    compiler_params=pltpu.CompilerParams(
        use_tc_tiling_on_sc=False, needs_layout_passes=False),
)(_sc_agg_body)


def _tc_post_body(S0_ref, S1_ref, den_ref, bias_ref, out_ref):
    den = den_ref[...] + 1e-16
    out_ref[:, :D2] = jnp.tanh(S0_ref[...] / den + bias_ref[:, :D2])
    out_ref[:, D2:] = jnp.tanh(S1_ref[...] / den + bias_ref[:, D2:])


def _tc_post(S0, S1, den_col, bias2):
    return pl.pallas_call(
        _tc_post_body,
        out_shape=jax.ShapeDtypeStruct((N, D), jnp.float32),
    )(S0, S1, den_col, bias2)


def kernel(x, edge_index, W, att_src, att_dst, bias):
    src = edge_index[0].astype(jnp.int32)
    dst = edge_index[1].astype(jnp.int32)
    att2 = jnp.stack([att_src, att_dst], axis=1)
    h, asd = _tc_pre(x, W, att2)
    ht = h.reshape(2 * N, D2)
    S0, S1, den_p = _sc_agg(ht, asd, src, dst)
    return _tc_post(S0, S1, den_p.reshape(N, 1), bias.reshape(1, D))
